# Initial kernel scaffold; baseline (speedup 1.0000x reference)
import jax
import jax.numpy as jnp
from jax.experimental import pallas as pl


def kernel(x, weightA, weightB, weightC, bias):
    raise NotImplementedError("write your pallas kernel here")


# speedup vs baseline: 2.7573x; 2.7573x over previous
import jax
import jax.numpy as jnp
from jax import lax
from jax.experimental import pallas as pl
from jax.experimental.pallas import tpu as pltpu


def _round_up(v, m):
    return ((v + m - 1) // m) * m


def _pad2(a, rows, cols):
    pr, pc = rows - a.shape[0], cols - a.shape[1]
    if pr or pc:
        a = jnp.pad(a, ((0, pr), (0, pc)))
    return a


def _fused_body(x_ref, a_ref, c_ref, b_ref, bias_ref, o_ref, xa_ref):
    j = pl.program_id(1)

    @pl.when(j == 0)
    def _project():
        xa_ref[...] = lax.dot_general(
            x_ref[...], a_ref[...],
            dimension_numbers=(((1,), (1,)), ((), ())),
            preferred_element_type=jnp.float32,
        ).astype(jnp.bfloat16)

    dense = lax.dot_general(
        x_ref[...], c_ref[...],
        dimension_numbers=(((1,), (1,)), ((), ())),
        preferred_element_type=jnp.float32,
    )
    low = lax.dot_general(
        xa_ref[...], b_ref[...],
        dimension_numbers=(((1,), (1,)), ((), ())),
        preferred_element_type=jnp.float32,
    )
    o_ref[...] = dense + low + bias_ref[...]


def kernel(x, weightA, weightB, weightC, bias):
    batch, in_f = x.shape
    out_f, rank = weightB.shape
    out_dtype = x.dtype

    tm = min(1024, _round_up(batch, 8))
    tn = min(1024, _round_up(out_f, 128))
    M = _round_up(batch, tm)
    N = _round_up(out_f, tn)
    K = _round_up(in_f, 128)
    R = _round_up(rank, 128)

    x_p = _pad2(x, M, K).astype(jnp.bfloat16)
    a_p = _pad2(weightA, R, K).astype(jnp.bfloat16)
    c_p = _pad2(weightC, N, K).astype(jnp.bfloat16)
    b_p = _pad2(weightB, N, R).astype(jnp.bfloat16)
    bias_p = _pad2(bias.reshape(1, out_f).astype(jnp.float32), 1, N)

    grid = (M // tm, N // tn)

    out = pl.pallas_call(
        _fused_body,
        out_shape=jax.ShapeDtypeStruct((M, N), out_dtype),
        grid=grid,
        in_specs=[
            pl.BlockSpec((tm, K), lambda i, j: (i, 0)),
            pl.BlockSpec((R, K), lambda i, j: (0, 0)),
            pl.BlockSpec((tn, K), lambda i, j: (j, 0)),
            pl.BlockSpec((tn, R), lambda i, j: (j, 0)),
            pl.BlockSpec((1, tn), lambda i, j: (0, j)),
        ],
        out_specs=pl.BlockSpec((tm, tn), lambda i, j: (i, j)),
        scratch_shapes=[
            pltpu.VMEM((tm, R), jnp.bfloat16),
        ],
        compiler_params=pltpu.CompilerParams(
            dimension_semantics=("parallel", "arbitrary"),
            vmem_limit_bytes=60 * 1024 * 1024,
        ),
    )(x_p, a_p, c_p, b_p, bias_p)

    if M != batch or N != out_f:
        out = out[:batch, :out_f]
    return out


# speedup vs baseline: 2.9593x; 1.0733x over previous
import jax
import jax.numpy as jnp
from jax import lax
from jax.experimental import pallas as pl
from jax.experimental.pallas import tpu as pltpu


def _round_up(v, m):
    return ((v + m - 1) // m) * m


def _pad2(a, rows, cols):
    pr, pc = rows - a.shape[0], cols - a.shape[1]
    if pr or pc:
        a = jnp.pad(a, ((0, pr), (0, pc)))
    return a


def _fused_body(x_ref, a_ref, c_ref, b_ref, bias_ref, o_ref, xs_ref, xa_ref):
    j = pl.program_id(1)

    @pl.when(j == 0)
    def _prep():
        xs_ref[...] = x_ref[...].astype(jnp.bfloat16)
        xa_ref[...] = lax.dot_general(
            xs_ref[...], a_ref[...],
            dimension_numbers=(((1,), (1,)), ((), ())),
            preferred_element_type=jnp.float32,
        ).astype(jnp.bfloat16)

    dense = lax.dot_general(
        xs_ref[...], c_ref[...],
        dimension_numbers=(((1,), (1,)), ((), ())),
        preferred_element_type=jnp.float32,
    )
    low = lax.dot_general(
        xa_ref[...], b_ref[...],
        dimension_numbers=(((1,), (1,)), ((), ())),
        preferred_element_type=jnp.float32,
    )
    o_ref[...] = dense + low + bias_ref[...]


def kernel(x, weightA, weightB, weightC, bias):
    batch, in_f = x.shape
    out_f, rank = weightB.shape
    out_dtype = x.dtype

    tm = min(1024, _round_up(batch, 8))
    tn = min(512, _round_up(out_f, 128))
    M = _round_up(batch, tm)
    N = _round_up(out_f, tn)
    K = _round_up(in_f, 128)
    R = _round_up(rank, 128)

    x_p = _pad2(x, M, K)
    a_p = _pad2(weightA, R, K).astype(jnp.bfloat16)
    c_p = _pad2(weightC, N, K).astype(jnp.bfloat16)
    b_p = _pad2(weightB, N, R).astype(jnp.bfloat16)
    bias_p = _pad2(bias.reshape(1, out_f).astype(jnp.float32), 1, N)

    grid = (M // tm, N // tn)

    out = pl.pallas_call(
        _fused_body,
        out_shape=jax.ShapeDtypeStruct((M, N), out_dtype),
        grid=grid,
        in_specs=[
            pl.BlockSpec((tm, K), lambda i, j: (i, 0)),
            pl.BlockSpec((R, K), lambda i, j: (0, 0)),
            pl.BlockSpec((tn, K), lambda i, j: (j, 0)),
            pl.BlockSpec((tn, R), lambda i, j: (j, 0)),
            pl.BlockSpec((1, tn), lambda i, j: (0, j)),
        ],
        out_specs=pl.BlockSpec((tm, tn), lambda i, j: (i, j)),
        scratch_shapes=[
            pltpu.VMEM((tm, K), jnp.bfloat16),
            pltpu.VMEM((tm, R), jnp.bfloat16),
        ],
        compiler_params=pltpu.CompilerParams(
            dimension_semantics=("parallel", "arbitrary"),
            vmem_limit_bytes=60 * 1024 * 1024,
        ),
    )(x_p, a_p, c_p, b_p, bias_p)

    if M != batch or N != out_f:
        out = out[:batch, :out_f]
    return out


# speedup vs baseline: 3.1671x; 1.0702x over previous
import jax
import jax.numpy as jnp
from jax import lax
from jax.experimental import pallas as pl
from jax.experimental.pallas import tpu as pltpu


def _round_up(v, m):
    return ((v + m - 1) // m) * m


def _pad2(a, rows, cols):
    pr, pc = rows - a.shape[0], cols - a.shape[1]
    if pr or pc:
        a = jnp.pad(a, ((0, pr), (0, pc)))
    return a


def _weight_body(b_ref, a_ref, c_ref, w_ref, ab_ref):
    n = pl.program_id(0)

    @pl.when(n == 0)
    def _prep():
        ab_ref[...] = a_ref[...].astype(jnp.bfloat16)

    low = lax.dot_general(
        b_ref[...].astype(jnp.bfloat16), ab_ref[...],
        dimension_numbers=(((1,), (0,)), ((), ())),
        preferred_element_type=jnp.float32,
    )
    w_ref[...] = (c_ref[...] + low).astype(jnp.bfloat16)


def _gemm_body(x_ref, w_ref, bias_ref, o_ref, xs_ref):
    j = pl.program_id(1)

    @pl.when(j == 0)
    def _cast_x():
        xs_ref[...] = x_ref[...].astype(jnp.bfloat16)

    o_ref[...] = lax.dot_general(
        xs_ref[...], w_ref[...],
        dimension_numbers=(((1,), (1,)), ((), ())),
        preferred_element_type=jnp.float32,
    ) + bias_ref[...]


def kernel(x, weightA, weightB, weightC, bias):
    batch, in_f = x.shape
    out_f, rank = weightB.shape
    out_dtype = x.dtype

    tm = min(1024, _round_up(batch, 8))
    tn = min(512, _round_up(out_f, 128))
    tw = min(512, _round_up(out_f, 128))
    M = _round_up(batch, tm)
    N = _round_up(out_f, tn)
    K = _round_up(in_f, 128)
    R = _round_up(rank, 128)

    x_p = _pad2(x, M, K)
    a_p = _pad2(weightA, R, K)
    c_p = _pad2(weightC, N, K)
    b_p = _pad2(weightB, N, R)
    bias_p = _pad2(bias.reshape(1, out_f).astype(jnp.float32), 1, N)

    w_eff = pl.pallas_call(
        _weight_body,
        out_shape=jax.ShapeDtypeStruct((N, K), jnp.bfloat16),
        grid=(N // tw,),
        in_specs=[
            pl.BlockSpec((tw, R), lambda n: (n, 0)),
            pl.BlockSpec((R, K), lambda n: (0, 0)),
            pl.BlockSpec((tw, K), lambda n: (n, 0)),
        ],
        out_specs=pl.BlockSpec((tw, K), lambda n: (n, 0)),
        scratch_shapes=[
            pltpu.VMEM((R, K), jnp.bfloat16),
        ],
        compiler_params=pltpu.CompilerParams(
            dimension_semantics=("arbitrary",),
            vmem_limit_bytes=56 * 1024 * 1024,
        ),
    )(b_p, a_p, c_p)

    out = pl.pallas_call(
        _gemm_body,
        out_shape=jax.ShapeDtypeStruct((M, N), out_dtype),
        grid=(M // tm, N // tn),
        in_specs=[
            pl.BlockSpec((tm, K), lambda i, j: (i, 0)),
            pl.BlockSpec((tn, K), lambda i, j: (j, 0)),
            pl.BlockSpec((1, tn), lambda i, j: (0, j)),
        ],
        out_specs=pl.BlockSpec((tm, tn), lambda i, j: (i, j)),
        scratch_shapes=[
            pltpu.VMEM((tm, K), jnp.bfloat16),
        ],
        compiler_params=pltpu.CompilerParams(
            dimension_semantics=("parallel", "arbitrary"),
            vmem_limit_bytes=56 * 1024 * 1024,
        ),
    )(x_p, w_eff, bias_p)

    if M != batch or N != out_f:
        out = out[:batch, :out_f]
    return out
